# Initial kernel scaffold; baseline (speedup 1.0000x reference)
import functools

import jax
import jax.numpy as jnp
from jax import lax
from jax.experimental import pallas as pl
from jax.experimental.pallas import tpu as pltpu
from jax.experimental.pallas import tpu_sc as plsc

N = 10000
E = 160000
D = 256
HD = 128
NC = 2
NS = 16
L = 16
NR = 640
E_PAD = 161792
EPT = E_PAD // NS
ECH = 128
NCH = EPT // ECH
DEG_CH = 1264
ESE = E_PAD // (NC * NS)
RPT = NR // NS
ARPT = N // NS

_vsm = plsc.VectorSubcoreMesh(
    core_axis_name="c", subcore_axis_name="s", num_cores=NC, num_subcores=NS)


def _nrsqrt(x):
    xi = plsc.bitcast(x, jnp.int32)
    y = plsc.bitcast(jnp.int32(0x5F3759DF) - (xi >> 1), jnp.float32)
    for _ in range(3):
        y = y * (1.5 - 0.5 * x * y * y)
    return y


@functools.partial(
    pl.kernel,
    out_type=(jax.ShapeDtypeStruct((NR, L), jnp.float32),
              jax.ShapeDtypeStruct((E_PAD,), jnp.float32)),
    mesh=_vsm,
    scratch_types=[
        pltpu.VMEM((NR, L), jnp.float32),
        pltpu.VMEM((DEG_CH,), jnp.int32),
        pltpu.VMEM((DEG_CH,), jnp.float32),
        pltpu.VMEM((DEG_CH,), jnp.float32),
        pltpu.VMEM((5, 128), jnp.int32),
        pltpu.VMEM((RPT, L), jnp.float32),
        pltpu.VMEM((NR, L), jnp.float32),
        pltpu.VMEM_SHARED((NR, L), jnp.float32),
        pltpu.VMEM_SHARED((NR, L), jnp.float32),
    ],
)
def _prep(src_hbm, dst_hbm, ew_hbm, dinv_hbm, se_hbm,
          deg_loc, ebuf_i, ebuf_f, se_buf, idx2d, dinv_loc, dinv_all,
          deg_sh, dinv_sh):
    c = lax.axis_index("c")
    s = lax.axis_index("s")
    zeros = jnp.zeros((L,), jnp.float32)

    def z_body(i, _):
        deg_loc[i, :] = zeros
        return 0
    lax.fori_loop(0, NR, z_body, 0)

    def z2_body(i, _):
        dinv_loc[i, :] = zeros
        return 0
    lax.fori_loop(0, RPT, z2_body, 0)

    def idx_body(i, _):
        j = i // 8
        k = i % 8
        idx2d[j, pl.ds(k * L, L)] = lax.iota(jnp.int32, L) + (j * 128 + k * L)
        return 0
    lax.fori_loop(0, NR // L, idx_body, 0)

    pltpu.sync_copy(dinv_loc, deg_sh.at[pl.ds(s * RPT, RPT)])
    plsc.subcore_barrier()

    def deg_chunk(ci, _):
        base = s * EPT + ci * DEG_CH
        pltpu.sync_copy(dst_hbm.at[pl.ds(base, DEG_CH)], ebuf_i)
        pltpu.sync_copy(ew_hbm.at[pl.ds(base, DEG_CH)], ebuf_f)

        def inner(i, _):
            dv = ebuf_i[pl.ds(i * L, L)]
            wv = ebuf_f[pl.ds(i * L, L)]
            plsc.addupdate_scatter(deg_loc, [dv >> 4, dv & 15], wv)
            return 0
        lax.fori_loop(0, DEG_CH // L, inner, 0)
        return 0
    lax.fori_loop(0, EPT // DEG_CH, deg_chunk, 0)

    for j in range(NR // 128):
        pltpu.sync_copy(deg_loc.at[pl.ds(j * 128, 128)],
                        deg_sh.at[idx2d.at[j]], add=True)
    plsc.subcore_barrier()

    pltpu.sync_copy(deg_sh.at[pl.ds(s * RPT, RPT)], dinv_loc)

    def dinv_body(i, _):
        dinv_loc[i, :] = _nrsqrt(dinv_loc[i, :] + 1.0)
        return 0
    lax.fori_loop(0, RPT, dinv_body, 0)
    pltpu.sync_copy(dinv_loc, dinv_sh.at[pl.ds(s * RPT, RPT)])

    @pl.when(c == 0)
    def _():
        pltpu.sync_copy(dinv_loc, dinv_hbm.at[pl.ds(s * RPT, RPT)])
    plsc.subcore_barrier()

    pltpu.sync_copy(dinv_sh, dinv_all)
    w = c * NS + s

    def se_chunk(ci, _):
        base = w * ESE + ci * DEG_CH
        pltpu.sync_copy(src_hbm.at[pl.ds(base, DEG_CH)], ebuf_i)
        pltpu.sync_copy(ew_hbm.at[pl.ds(base, DEG_CH)], ebuf_f)

        def inner(i, _):
            sv = ebuf_i[pl.ds(i * L, L)]
            dvv = plsc.load_gather(dinv_all, [sv >> 4, sv & 15])
            se_buf[pl.ds(i * L, L)] = ebuf_f[pl.ds(i * L, L)] * dvv
            return 0
        lax.fori_loop(0, DEG_CH // L, inner, 0)
        pltpu.sync_copy(se_buf, se_hbm.at[pl.ds(base, DEG_CH)])
        return 0
    lax.fori_loop(0, ESE // DEG_CH, se_chunk, 0)


@functools.partial(
    pl.kernel,
    out_type=(jax.ShapeDtypeStruct((N, HD), jnp.float32),
              jax.ShapeDtypeStruct((N, HD), jnp.float32)),
    mesh=_vsm,
    scratch_types=[
        pltpu.VMEM((ECH, HD), jnp.float32),
        pltpu.VMEM((ECH,), jnp.int32),
        pltpu.VMEM((1, ECH), jnp.int32),
        pltpu.VMEM((ECH,), jnp.float32),
        pltpu.VMEM((125, HD), jnp.float32),
        pltpu.SemaphoreType.DMA,
        pltpu.VMEM_SHARED((N, HD), jnp.float32),
    ],
)
def _agg(h0, h1, src_hbm, dst_hbm, se_hbm, o0, o1,
         gbuf, src_buf, dst_buf, se_buf, zbuf, sem, acc_sh):
    c = lax.axis_index("c")
    s = lax.axis_index("s")
    zeros = jnp.zeros((L,), jnp.float32)

    def zb(i, _):
        for k in range(HD // L):
            zbuf[i, pl.ds(k * L, L)] = zeros
        return 0
    lax.fori_loop(0, 125, zb, 0)
    for r in range(ARPT // 125):
        pltpu.sync_copy(zbuf, acc_sh.at[pl.ds(s * ARPT + r * 125, 125)])
    plsc.subcore_barrier()

    def run_half(h_hbm):
        def chunk(j, _):
            base = s * EPT + j * ECH
            pltpu.sync_copy(src_hbm.at[pl.ds(base, ECH)], src_buf)
            pltpu.sync_copy(dst_hbm.at[pl.ds(base, ECH)], dst_buf.at[0])
            pltpu.sync_copy(se_hbm.at[pl.ds(base, ECH)], se_buf)
            pltpu.async_copy(h_hbm.at[src_buf], gbuf, sem).wait()

            def edge(e, _):
                sv = plsc.load_gather(se_buf, [jnp.full((L,), e, jnp.int32)])
                for k in range(HD // L):
                    g = gbuf[e, pl.ds(k * L, L)]
                    gbuf[e, pl.ds(k * L, L)] = g * sv
                return 0
            lax.fori_loop(0, ECH, edge, 0)
            pltpu.sync_copy(gbuf, acc_sh.at[dst_buf.at[0]], add=True)
            return 0
        lax.fori_loop(0, NCH, chunk, 0)

    @pl.when(c == 0)
    def _():
        run_half(h0)

    @pl.when(c == 1)
    def _():
        run_half(h1)

    plsc.subcore_barrier()

    @pl.when(c == 0)
    def _():
        pltpu.sync_copy(acc_sh.at[pl.ds(s * ARPT, ARPT)],
                        o0.at[pl.ds(s * ARPT, ARPT)])

    @pl.when(c == 1)
    def _():
        pltpu.sync_copy(acc_sh.at[pl.ds(s * ARPT, ARPT)],
                        o1.at[pl.ds(s * ARPT, ARPT)])


def _mm_body(x_ref, w_ref, o0_ref, o1_ref):
    h = jnp.dot(x_ref[...], w_ref[...], preferred_element_type=jnp.float32,
                precision=lax.Precision.HIGHEST)
    o0_ref[...] = h[:, :HD]
    o1_ref[...] = h[:, HD:]


_MMR = 1000


def _matmul_split(x, w):
    return pl.pallas_call(
        _mm_body,
        grid=(N // _MMR,),
        in_specs=[pl.BlockSpec((_MMR, D), lambda i: (i, 0)),
                  pl.BlockSpec((D, D), lambda i: (0, 0))],
        out_specs=[pl.BlockSpec((_MMR, HD), lambda i: (i, 0)),
                   pl.BlockSpec((_MMR, HD), lambda i: (i, 0))],
        out_shape=[jax.ShapeDtypeStruct((N, HD), jnp.float32),
                   jax.ShapeDtypeStruct((N, HD), jnp.float32)],
    )(x, w)


def _epi_body(a0_ref, a1_ref, h0_ref, h1_ref, dv_ref, b_ref, o_ref):
    dv = dv_ref[...]
    dv2 = dv * dv
    b = b_ref[...]
    m0 = dv * a0_ref[...] + dv2 * h0_ref[...] + b[:, :HD]
    m1 = dv * a1_ref[...] + dv2 * h1_ref[...] + b[:, HD:]
    o_ref[:, :HD] = jnp.maximum(m0, 0.0)
    o_ref[:, HD:] = jnp.maximum(m1, 0.0)


def _epilogue(a0, a1, h0, h1, dinv_col, b_row):
    return pl.pallas_call(
        _epi_body,
        grid=(N // _MMR,),
        in_specs=[pl.BlockSpec((_MMR, HD), lambda i: (i, 0)),
                  pl.BlockSpec((_MMR, HD), lambda i: (i, 0)),
                  pl.BlockSpec((_MMR, HD), lambda i: (i, 0)),
                  pl.BlockSpec((_MMR, HD), lambda i: (i, 0)),
                  pl.BlockSpec((_MMR, 1), lambda i: (i, 0)),
                  pl.BlockSpec((1, D), lambda i: (0, 0))],
        out_specs=pl.BlockSpec((_MMR, D), lambda i: (i, 0)),
        out_shape=jax.ShapeDtypeStruct((N, D), jnp.float32),
    )(a0, a1, h0, h1, dinv_col, b_row)


def kernel(X, edge_index, edge_weight, W1, b1, W2, b2):
    src = edge_index[0]
    dst = edge_index[1]
    pad_i = jnp.zeros((E_PAD - E,), jnp.int32)
    srcp = jnp.concatenate([src, pad_i])
    dstp = jnp.concatenate([dst, pad_i])
    ewp = jnp.concatenate([edge_weight, jnp.zeros((E_PAD - E,), jnp.float32)])

    dinv2d, sep = _prep(srcp, dstp, ewp)
    dinv_col = dinv2d.reshape(-1)[:N].reshape(N, 1)
    b1r = b1.reshape(1, D)
    b2r = b2.reshape(1, D)

    h1a, h1b = _matmul_split(X, W1)
    a1a, a1b = _agg(h1a, h1b, srcp, dstp, sep)
    out1 = _epilogue(a1a, a1b, h1a, h1b, dinv_col, b1r)

    h2a, h2b = _matmul_split(out1, W2)
    a2a, a2b = _agg(h2a, h2b, srcp, dstp, sep)
    return _epilogue(a2a, a2b, h2a, h2b, dinv_col, b2r)


# speedup vs baseline: 4.6297x; 4.6297x over previous
import functools

import jax
import jax.numpy as jnp
from jax import lax
from jax.experimental import pallas as pl
from jax.experimental.pallas import tpu as pltpu
from jax.experimental.pallas import tpu_sc as plsc

N = 10000
E = 160000
D = 256
HD = 128
NC = 2
NS = 16
L = 16
E_PAD = 163840
EPT = E_PAD // NS
ECH = 128
NCH = EPT // ECH
DEG_CH = 1024
ESE = E_PAD // (NC * NS)
DRN = 632
DRL = 520
ZRN = 640
ZRL = 400

_vsm = plsc.VectorSubcoreMesh(
    core_axis_name="c", subcore_axis_name="s", num_cores=NC, num_subcores=NS)


def _nrsqrt(x):
    xi = plsc.bitcast(x, jnp.int32)
    y = plsc.bitcast(jnp.int32(0x5F3759DF) - (xi >> 1), jnp.float32)
    for _ in range(3):
        y = y * (1.5 - 0.5 * x * y * y)
    return y


NP = 10240
NPT = NP // NS


@functools.partial(
    pl.kernel,
    out_type=(jax.ShapeDtypeStruct((NP,), jnp.float32),
              jax.ShapeDtypeStruct((E_PAD,), jnp.float32)),
    mesh=_vsm,
    scratch_types=[
        pltpu.VMEM((NP,), jnp.float32),
        pltpu.VMEM((DEG_CH,), jnp.int32),
        pltpu.VMEM((DEG_CH,), jnp.float32),
        pltpu.VMEM((DEG_CH,), jnp.float32),
        pltpu.VMEM((NS, NPT), jnp.float32),
        pltpu.VMEM((NPT,), jnp.float32),
        pltpu.VMEM((NP,), jnp.float32),
        pltpu.VMEM_SHARED((NS, NP), jnp.float32),
        pltpu.VMEM_SHARED((NP,), jnp.float32),
    ],
    compiler_params=pltpu.CompilerParams(needs_layout_passes=False),
)
def _prep(src_hbm, dst_hbm, ew_hbm, dinv_hbm, se_hbm,
          deg_loc, ebuf_i, ebuf_f, se_buf, d2buf, dinv_loc, dinv_all,
          deg_sh, dinv_sh):
    c = lax.axis_index("c")
    s = lax.axis_index("s")
    zeros = jnp.zeros((L,), jnp.float32)

    def z_body(i, _):
        deg_loc[pl.ds(i * L, L)] = zeros
        return 0
    lax.fori_loop(0, NP // L, z_body, 0)

    def deg_chunk(ci, _):
        base = s * EPT + ci * DEG_CH
        pltpu.sync_copy(dst_hbm.at[pl.ds(base, DEG_CH)], ebuf_i)
        pltpu.sync_copy(ew_hbm.at[pl.ds(base, DEG_CH)], ebuf_f)

        def inner(i, _):
            dv = ebuf_i[pl.ds(i * L, L)]
            wv = ebuf_f[pl.ds(i * L, L)]
            plsc.addupdate_scatter(deg_loc, [dv], wv)
            return 0
        lax.fori_loop(0, DEG_CH // L, inner, 0)
        return 0
    lax.fori_loop(0, EPT // DEG_CH, deg_chunk, 0)

    pltpu.sync_copy(deg_loc, deg_sh.at[s])
    plsc.subcore_barrier()
    pltpu.sync_copy(deg_sh.at[:, pl.ds(s * NPT, NPT)], d2buf)

    def red_body(i, _):
        acc = d2buf[0, pl.ds(i * L, L)]
        for r in range(1, NS):
            acc = acc + d2buf[r, pl.ds(i * L, L)]
        dinv_loc[pl.ds(i * L, L)] = _nrsqrt(acc + 1.0)
        return 0
    lax.fori_loop(0, NPT // L, red_body, 0)
    pltpu.sync_copy(dinv_loc, dinv_sh.at[pl.ds(s * NPT, NPT)])

    @pl.when(c == 0)
    def _():
        pltpu.sync_copy(dinv_loc, dinv_hbm.at[pl.ds(s * NPT, NPT)])
    plsc.subcore_barrier()

    pltpu.sync_copy(dinv_sh, dinv_all)
    w = c * NS + s

    def se_chunk(ci, _):
        base = w * ESE + ci * DEG_CH
        pltpu.sync_copy(src_hbm.at[pl.ds(base, DEG_CH)], ebuf_i)
        pltpu.sync_copy(ew_hbm.at[pl.ds(base, DEG_CH)], ebuf_f)

        def inner(i, _):
            sv = ebuf_i[pl.ds(i * L, L)]
            dvv = plsc.load_gather(dinv_all, [sv])
            se_buf[pl.ds(i * L, L)] = ebuf_f[pl.ds(i * L, L)] * dvv
            return 0
        lax.fori_loop(0, DEG_CH // L, inner, 0)
        pltpu.sync_copy(se_buf, se_hbm.at[pl.ds(base, DEG_CH)])
        return 0
    lax.fori_loop(0, ESE // DEG_CH, se_chunk, 0)


@functools.partial(
    pl.kernel,
    out_type=(jax.ShapeDtypeStruct((N, HD), jnp.float32),
              jax.ShapeDtypeStruct((N, HD), jnp.float32)),
    mesh=_vsm,
    scratch_types=[
        pltpu.VMEM((ECH, HD), jnp.float32),
        pltpu.VMEM((ECH,), jnp.int32),
        pltpu.VMEM((1, ECH), jnp.int32),
        pltpu.VMEM((ECH,), jnp.float32),
        pltpu.VMEM((80, HD), jnp.float32),
        pltpu.SemaphoreType.DMA,
        pltpu.VMEM_SHARED((N, HD), jnp.float32),
    ],
    compiler_params=pltpu.CompilerParams(needs_layout_passes=False),
)
def _agg(h0, h1, src_hbm, dst_hbm, se_hbm, o0, o1,
         gbuf, src_buf, dst_buf, se_buf, zbuf, sem, acc_sh):
    c = lax.axis_index("c")
    s = lax.axis_index("s")
    zeros = jnp.zeros((L,), jnp.float32)

    def zb(i, _):
        for k in range(HD // L):
            zbuf[i, pl.ds(k * L, L)] = zeros
        return 0
    lax.fori_loop(0, 80, zb, 0)

    @pl.when(s < NS - 1)
    def _():
        for r in range(ZRN // 80):
            pltpu.sync_copy(zbuf, acc_sh.at[pl.ds(s * ZRN + r * 80, 80)])

    @pl.when(s == NS - 1)
    def _():
        for r in range(ZRL // 80):
            pltpu.sync_copy(zbuf, acc_sh.at[pl.ds((NS - 1) * ZRN + r * 80, 80)])
    plsc.subcore_barrier()

    def run_half(h_hbm):
        def chunk(j, _):
            base = s * EPT + j * ECH
            pltpu.sync_copy(src_hbm.at[pl.ds(base, ECH)], src_buf)
            pltpu.sync_copy(dst_hbm.at[pl.ds(base, ECH)], dst_buf.at[0])
            pltpu.sync_copy(se_hbm.at[pl.ds(base, ECH)], se_buf)
            pltpu.async_copy(h_hbm.at[src_buf], gbuf, sem).wait()

            def edge(e, _):
                sv = plsc.load_gather(se_buf, [jnp.full((L,), e, jnp.int32)])
                for k in range(HD // L):
                    g = gbuf[e, pl.ds(k * L, L)]
                    gbuf[e, pl.ds(k * L, L)] = g * sv
                return 0
            lax.fori_loop(0, ECH, edge, 0)
            pltpu.sync_copy(gbuf, acc_sh.at[dst_buf.at[0]], add=True)
            return 0
        lax.fori_loop(0, NCH, chunk, 0)

    @pl.when(c == 0)
    def _():
        run_half(h0)

    @pl.when(c == 1)
    def _():
        run_half(h1)

    plsc.subcore_barrier()

    def drain(o_hbm):
        @pl.when(s < NS - 1)
        def _():
            pltpu.sync_copy(acc_sh.at[pl.ds(s * DRN, DRN)],
                            o_hbm.at[pl.ds(s * DRN, DRN)])

        @pl.when(s == NS - 1)
        def _():
            pltpu.sync_copy(acc_sh.at[pl.ds((NS - 1) * DRN, DRL)],
                            o_hbm.at[pl.ds((NS - 1) * DRN, DRL)])

    @pl.when(c == 0)
    def _():
        drain(o0)

    @pl.when(c == 1)
    def _():
        drain(o1)


def _mm_body(x_ref, w_ref, o0_ref, o1_ref):
    h = jnp.dot(x_ref[...], w_ref[...], preferred_element_type=jnp.float32,
                precision=lax.Precision.HIGHEST)
    o0_ref[...] = h[:, :HD]
    o1_ref[...] = h[:, HD:]


_MMR = 1000


def _matmul_split(x, w):
    return pl.pallas_call(
        _mm_body,
        grid=(N // _MMR,),
        in_specs=[pl.BlockSpec((_MMR, D), lambda i: (i, 0)),
                  pl.BlockSpec((D, D), lambda i: (0, 0))],
        out_specs=[pl.BlockSpec((_MMR, HD), lambda i: (i, 0)),
                   pl.BlockSpec((_MMR, HD), lambda i: (i, 0))],
        out_shape=[jax.ShapeDtypeStruct((N, HD), jnp.float32),
                   jax.ShapeDtypeStruct((N, HD), jnp.float32)],
    )(x, w)


def _epi_body(a0_ref, a1_ref, h0_ref, h1_ref, dv_ref, b_ref, o_ref):
    dv = dv_ref[...]
    dv2 = dv * dv
    b = b_ref[...]
    m0 = dv * a0_ref[...] + dv2 * h0_ref[...] + b[:, :HD]
    m1 = dv * a1_ref[...] + dv2 * h1_ref[...] + b[:, HD:]
    o_ref[:, :HD] = jnp.maximum(m0, 0.0)
    o_ref[:, HD:] = jnp.maximum(m1, 0.0)


def _epilogue(a0, a1, h0, h1, dinv_col, b_row):
    return pl.pallas_call(
        _epi_body,
        grid=(N // _MMR,),
        in_specs=[pl.BlockSpec((_MMR, HD), lambda i: (i, 0)),
                  pl.BlockSpec((_MMR, HD), lambda i: (i, 0)),
                  pl.BlockSpec((_MMR, HD), lambda i: (i, 0)),
                  pl.BlockSpec((_MMR, HD), lambda i: (i, 0)),
                  pl.BlockSpec((_MMR, 1), lambda i: (i, 0)),
                  pl.BlockSpec((1, D), lambda i: (0, 0))],
        out_specs=pl.BlockSpec((_MMR, D), lambda i: (i, 0)),
        out_shape=jax.ShapeDtypeStruct((N, D), jnp.float32),
    )(a0, a1, h0, h1, dinv_col, b_row)


def kernel(X, edge_index, edge_weight, W1, b1, W2, b2):
    src = edge_index[0]
    dst = edge_index[1]
    pad_i = jnp.zeros((E_PAD - E,), jnp.int32)
    srcp = jnp.concatenate([src, pad_i])
    dstp = jnp.concatenate([dst, pad_i])
    ewp = jnp.concatenate([edge_weight, jnp.zeros((E_PAD - E,), jnp.float32)])

    dinv1d, sep = _prep(srcp, dstp, ewp)
    dinv_col = dinv1d[:N].reshape(N, 1)
    b1r = b1.reshape(1, D)
    b2r = b2.reshape(1, D)

    h1a, h1b = _matmul_split(X, W1)
    a1a, a1b = _agg(h1a, h1b, srcp, dstp, sep)
    out1 = _epilogue(a1a, a1b, h1a, h1b, dinv_col, b1r)

    h2a, h2b = _matmul_split(out1, W2)
    a2a, a2b = _agg(h2a, h2b, srcp, dstp, sep)
    return _epilogue(a2a, a2b, h2a, h2b, dinv_col, b2r)


# speedup vs baseline: 7.1631x; 1.5472x over previous
import functools

import jax
import jax.numpy as jnp
from jax import lax
from jax.experimental import pallas as pl
from jax.experimental.pallas import tpu as pltpu
from jax.experimental.pallas import tpu_sc as plsc

N = 10000
E = 160000
D = 256
HD = 128
NC = 2
NS = 16
L = 16
E_PAD = 163840
EPT = E_PAD // NS
ECH = 128
NCH = EPT // ECH
DEG_CH = 1024
ESE = E_PAD // (NC * NS)
DRN = 632
DRL = 520
ZRN = 640
ZRL = 400

_vsm = plsc.VectorSubcoreMesh(
    core_axis_name="c", subcore_axis_name="s", num_cores=NC, num_subcores=NS)


def _nrsqrt(x):
    xi = plsc.bitcast(x, jnp.int32)
    y = plsc.bitcast(jnp.int32(0x5F3759DF) - (xi >> 1), jnp.float32)
    for _ in range(3):
        y = y * (1.5 - 0.5 * x * y * y)
    return y


NP = 10240
NPT = NP // NS


@functools.partial(
    pl.kernel,
    out_type=(jax.ShapeDtypeStruct((NP,), jnp.float32),
              jax.ShapeDtypeStruct((E_PAD,), jnp.float32)),
    mesh=_vsm,
    scratch_types=[
        pltpu.VMEM((NP,), jnp.float32),
        pltpu.VMEM((DEG_CH,), jnp.int32),
        pltpu.VMEM((DEG_CH,), jnp.float32),
        pltpu.VMEM((DEG_CH,), jnp.float32),
        pltpu.VMEM((NS, NPT), jnp.float32),
        pltpu.VMEM((NPT,), jnp.float32),
        pltpu.VMEM((NP,), jnp.float32),
        pltpu.VMEM_SHARED((NS, NP), jnp.float32),
        pltpu.VMEM_SHARED((NP,), jnp.float32),
    ],
    compiler_params=pltpu.CompilerParams(needs_layout_passes=False),
)
def _prep(src_hbm, dst_hbm, ew_hbm, dinv_hbm, se_hbm,
          deg_loc, ebuf_i, ebuf_f, se_buf, d2buf, dinv_loc, dinv_all,
          deg_sh, dinv_sh):
    c = lax.axis_index("c")
    s = lax.axis_index("s")
    zeros = jnp.zeros((L,), jnp.float32)

    def z_body(i, _):
        deg_loc[pl.ds(i * L, L)] = zeros
        return 0
    lax.fori_loop(0, NP // L, z_body, 0)

    def deg_chunk(ci, _):
        base = s * EPT + ci * DEG_CH
        pltpu.sync_copy(dst_hbm.at[pl.ds(base, DEG_CH)], ebuf_i)
        pltpu.sync_copy(ew_hbm.at[pl.ds(base, DEG_CH)], ebuf_f)

        def inner(i, _):
            dv = ebuf_i[pl.ds(i * L, L)]
            wv = ebuf_f[pl.ds(i * L, L)]
            plsc.addupdate_scatter(deg_loc, [dv], wv)
            return 0
        lax.fori_loop(0, DEG_CH // L, inner, 0)
        return 0
    lax.fori_loop(0, EPT // DEG_CH, deg_chunk, 0)

    pltpu.sync_copy(deg_loc, deg_sh.at[s])
    plsc.subcore_barrier()
    pltpu.sync_copy(deg_sh.at[:, pl.ds(s * NPT, NPT)], d2buf)

    def red_body(i, _):
        acc = d2buf[0, pl.ds(i * L, L)]
        for r in range(1, NS):
            acc = acc + d2buf[r, pl.ds(i * L, L)]
        dinv_loc[pl.ds(i * L, L)] = _nrsqrt(acc + 1.0)
        return 0
    lax.fori_loop(0, NPT // L, red_body, 0)
    pltpu.sync_copy(dinv_loc, dinv_sh.at[pl.ds(s * NPT, NPT)])

    @pl.when(c == 0)
    def _():
        pltpu.sync_copy(dinv_loc, dinv_hbm.at[pl.ds(s * NPT, NPT)])
    plsc.subcore_barrier()

    pltpu.sync_copy(dinv_sh, dinv_all)
    w = c * NS + s

    def se_chunk(ci, _):
        base = w * ESE + ci * DEG_CH
        pltpu.sync_copy(src_hbm.at[pl.ds(base, DEG_CH)], ebuf_i)
        pltpu.sync_copy(ew_hbm.at[pl.ds(base, DEG_CH)], ebuf_f)

        def inner(i, _):
            sv = ebuf_i[pl.ds(i * L, L)]
            dvv = plsc.load_gather(dinv_all, [sv])
            se_buf[pl.ds(i * L, L)] = ebuf_f[pl.ds(i * L, L)] * dvv
            return 0
        lax.fori_loop(0, DEG_CH // L, inner, 0)
        pltpu.sync_copy(se_buf, se_hbm.at[pl.ds(base, DEG_CH)])
        return 0
    lax.fori_loop(0, ESE // DEG_CH, se_chunk, 0)


@functools.partial(
    pl.kernel,
    out_type=(jax.ShapeDtypeStruct((N, HD), jnp.float32),
              jax.ShapeDtypeStruct((N, HD), jnp.float32)),
    mesh=_vsm,
    scratch_types=[
        pltpu.VMEM((2, ECH, HD), jnp.float32),
        pltpu.VMEM((3, ECH), jnp.int32),
        pltpu.VMEM((3, ECH), jnp.int32),
        pltpu.VMEM((3, ECH), jnp.float32),
        pltpu.VMEM((ECH,), jnp.float32),
        pltpu.VMEM((80, HD), jnp.float32),
        pltpu.SemaphoreType.DMA,
        pltpu.SemaphoreType.DMA,
        pltpu.SemaphoreType.DMA,
        pltpu.VMEM_SHARED((N, HD), jnp.float32),
    ],
    compiler_params=pltpu.CompilerParams(needs_layout_passes=False),
)
def _agg(h0, h1, src_hbm, dst_hbm, se_hbm, o0, o1,
         gbuf3, src3b, dst3b, se3b, se_buf, zbuf, gsem, ssem, isem, acc_sh):
    c = lax.axis_index("c")
    s = lax.axis_index("s")
    zeros = jnp.zeros((L,), jnp.float32)

    def zb(i, _):
        for k in range(HD // L):
            zbuf[i, pl.ds(k * L, L)] = zeros
        return 0
    lax.fori_loop(0, 80, zb, 0)

    @pl.when(s < NS - 1)
    def _():
        for r in range(ZRN // 80):
            pltpu.sync_copy(zbuf, acc_sh.at[pl.ds(s * ZRN + r * 80, 80)])

    @pl.when(s == NS - 1)
    def _():
        for r in range(ZRL // 80):
            pltpu.sync_copy(zbuf, acc_sh.at[pl.ds((NS - 1) * ZRN + r * 80, 80)])
    plsc.subcore_barrier()

    def run_half(h_hbm):
        def start_idx(j):
            r = lax.rem(j, 3)
            pltpu.async_copy(src_hbm.at[s, j], src3b.at[r], isem)
            pltpu.async_copy(dst_hbm.at[s, j], dst3b.at[r], isem)
            pltpu.async_copy(se_hbm.at[s, j], se3b.at[r], isem)

        def drain_idx():
            pltpu.make_async_copy(src_hbm.at[s, 0], src3b.at[0], isem).wait()
            pltpu.make_async_copy(dst_hbm.at[s, 0], dst3b.at[0], isem).wait()
            pltpu.make_async_copy(se_hbm.at[s, 0], se3b.at[0], isem).wait()

        def drain_row(sem):
            pltpu.make_async_copy(
                h_hbm.at[pl.ds(0, ECH)], gbuf3.at[0], sem).wait()

        start_idx(0)
        start_idx(1)
        drain_idx()
        pltpu.async_copy(h_hbm.at[src3b.at[0]], gbuf3.at[0], gsem)

        def chunk(j, _):
            jb = lax.rem(j, 2)
            nb = lax.rem(j + 1, 2)
            jr = lax.rem(j, 3)

            @pl.when(j + 1 < NCH)
            def _():
                drain_idx()

            @pl.when(j + 2 < NCH)
            def _():
                start_idx(j + 2)

            @pl.when(j >= 1)
            def _():
                drain_row(ssem)

            @pl.when(j + 1 < NCH)
            def _():
                pltpu.async_copy(
                    h_hbm.at[src3b.at[lax.rem(j + 1, 3)]], gbuf3.at[nb], gsem)

            drain_row(gsem)

            for k in range(ECH // L):
                se_buf[pl.ds(k * L, L)] = se3b[jr, pl.ds(k * L, L)]

            def edge(e, _):
                sv = plsc.load_gather(se_buf, [jnp.full((L,), e, jnp.int32)])
                for k in range(HD // L):
                    g = gbuf3[jb, e, pl.ds(k * L, L)]
                    gbuf3[jb, e, pl.ds(k * L, L)] = g * sv
                return 0
            lax.fori_loop(0, ECH, edge, 0)
            pltpu.async_copy(gbuf3.at[jb], acc_sh.at[dst3b.at[jr]],
                             ssem, add=True)
            return 0
        lax.fori_loop(0, NCH, chunk, 0)
        drain_row(ssem)

    @pl.when(c == 0)
    def _():
        run_half(h0)

    @pl.when(c == 1)
    def _():
        run_half(h1)

    plsc.subcore_barrier()

    def drain(o_hbm):
        @pl.when(s < NS - 1)
        def _():
            pltpu.sync_copy(acc_sh.at[pl.ds(s * DRN, DRN)],
                            o_hbm.at[pl.ds(s * DRN, DRN)])

        @pl.when(s == NS - 1)
        def _():
            pltpu.sync_copy(acc_sh.at[pl.ds((NS - 1) * DRN, DRL)],
                            o_hbm.at[pl.ds((NS - 1) * DRN, DRL)])

    @pl.when(c == 0)
    def _():
        drain(o0)

    @pl.when(c == 1)
    def _():
        drain(o1)


def _mm_body(x_ref, w_ref, o0_ref, o1_ref):
    h = jnp.dot(x_ref[...], w_ref[...], preferred_element_type=jnp.float32,
                precision=lax.Precision.HIGHEST)
    o0_ref[...] = h[:, :HD]
    o1_ref[...] = h[:, HD:]


_MMR = 1000


def _matmul_split(x, w):
    return pl.pallas_call(
        _mm_body,
        grid=(N // _MMR,),
        in_specs=[pl.BlockSpec((_MMR, D), lambda i: (i, 0)),
                  pl.BlockSpec((D, D), lambda i: (0, 0))],
        out_specs=[pl.BlockSpec((_MMR, HD), lambda i: (i, 0)),
                   pl.BlockSpec((_MMR, HD), lambda i: (i, 0))],
        out_shape=[jax.ShapeDtypeStruct((N, HD), jnp.float32),
                   jax.ShapeDtypeStruct((N, HD), jnp.float32)],
    )(x, w)


def _epi_body(a0_ref, a1_ref, h0_ref, h1_ref, dv_ref, b_ref, o_ref):
    dv = dv_ref[...]
    dv2 = dv * dv
    b = b_ref[...]
    m0 = dv * a0_ref[...] + dv2 * h0_ref[...] + b[:, :HD]
    m1 = dv * a1_ref[...] + dv2 * h1_ref[...] + b[:, HD:]
    o_ref[:, :HD] = jnp.maximum(m0, 0.0)
    o_ref[:, HD:] = jnp.maximum(m1, 0.0)


def _epilogue(a0, a1, h0, h1, dinv_col, b_row):
    return pl.pallas_call(
        _epi_body,
        grid=(N // _MMR,),
        in_specs=[pl.BlockSpec((_MMR, HD), lambda i: (i, 0)),
                  pl.BlockSpec((_MMR, HD), lambda i: (i, 0)),
                  pl.BlockSpec((_MMR, HD), lambda i: (i, 0)),
                  pl.BlockSpec((_MMR, HD), lambda i: (i, 0)),
                  pl.BlockSpec((_MMR, 1), lambda i: (i, 0)),
                  pl.BlockSpec((1, D), lambda i: (0, 0))],
        out_specs=pl.BlockSpec((_MMR, D), lambda i: (i, 0)),
        out_shape=jax.ShapeDtypeStruct((N, D), jnp.float32),
    )(a0, a1, h0, h1, dinv_col, b_row)


def kernel(X, edge_index, edge_weight, W1, b1, W2, b2):
    src = edge_index[0]
    dst = edge_index[1]
    pad_i = jnp.zeros((E_PAD - E,), jnp.int32)
    srcp = jnp.concatenate([src, pad_i])
    dstp = jnp.concatenate([dst, pad_i])
    ewp = jnp.concatenate([edge_weight, jnp.zeros((E_PAD - E,), jnp.float32)])

    dinv1d, sep = _prep(srcp, dstp, ewp)
    dinv_col = dinv1d[:N].reshape(N, 1)
    b1r = b1.reshape(1, D)
    b2r = b2.reshape(1, D)
    src3 = srcp.reshape(NS, NCH, ECH)
    dst3 = dstp.reshape(NS, NCH, ECH)
    se3 = sep.reshape(NS, NCH, ECH)

    h1a, h1b = _matmul_split(X, W1)
    a1a, a1b = _agg(h1a, h1b, src3, dst3, se3)
    out1 = _epilogue(a1a, a1b, h1a, h1b, dinv_col, b1r)

    h2a, h2b = _matmul_split(out1, W2)
    a2a, a2b = _agg(h2a, h2b, src3, dst3, se3)
    return _epilogue(a2a, a2b, h2a, h2b, dinv_col, b2r)
